# baseline (speedup 1.0000x reference)
import jax
import jax.numpy as jnp
from jax import lax
from jax.experimental import pallas as pl
from jax.experimental.pallas import tpu as pltpu
from jax.experimental.pallas import tpu_sc as plsc

N = 10000
E = 320000
D = 128
H = 8
HD = D // H

NC = 2
NS = 16
NW = NC * NS
NPAD = 10240
EP = E // NW
BLK = 64
EPP = 10240

NR = 2
RSZ = NPAD // NR
ACCROWS = RSZ + 8
ZROWS = RSZ // NS
LASTREAL = N - (NR - 1) * RSZ

_mesh = plsc.VectorSubcoreMesh(core_axis_name="c", subcore_axis_name="s")


def _deg_body(e0s, e0d, e1s, e1d, z1d, out, idx_v, acc_v):
    c = lax.axis_index("c")
    s = lax.axis_index("s")
    wid = c * NS + s
    ones = jnp.full((16,), 1.0, dtype=jnp.float32)
    for a, arr in enumerate([e0s, e0d, e1s, e1d]):
        pltpu.sync_copy(z1d, acc_v)
        pltpu.sync_copy(arr.at[pl.ds(wid * EP, EP)], idx_v)

        @pl.loop(0, EP // 16)
        def _(i):
            idx16 = idx_v[pl.ds(i * 16, 16)]
            plsc.addupdate_scatter(acc_v, [idx16], ones)

        pltpu.sync_copy(acc_v, out.at[a, c, s])


_deg_call = pl.kernel(
    _deg_body,
    out_type=jax.ShapeDtypeStruct((4, NC, NS, NPAD), jnp.float32),
    mesh=_mesh,
    scratch_types=[
        pltpu.VMEM((EP,), jnp.int32),
        pltpu.VMEM((NPAD,), jnp.float32),
    ],
    compiler_params=pltpu.CompilerParams(needs_layout_passes=False),
)


def _degsum_body(p_ref, o_ref):
    o_ref[...] = jnp.sum(p_ref[...], axis=1, keepdims=True)


_DC = 1280

_degsum_call = pl.pallas_call(
    _degsum_body,
    grid=(4, NPAD // _DC),
    in_specs=[pl.BlockSpec((1, NW, _DC), lambda a, i: (a, 0, i))],
    out_specs=pl.BlockSpec((1, 1, _DC), lambda a, i: (a, 0, i)),
    out_shape=jax.ShapeDtypeStruct((4, 1, NPAD), jnp.float32),
)


def _edge_body(h, srcf, dstf, zsmall, out,
               src_v, dst_v, l0s, l0d, l1s, l1d,
               cnt_v, buf0, buf1, buf2, sem, acc):
    c = lax.axis_index("c")
    s = lax.axis_index("s")
    wid = c * NS + s
    pltpu.sync_copy(srcf.at[wid], src_v)
    pltpu.sync_copy(dstf.at[wid], dst_v)
    pltpu.sync_copy(zsmall, acc.at[pl.ds(s * ZROWS, ZROWS)])

    zeros16 = jnp.zeros((16,), jnp.int32)
    trash16 = jnp.full((16,), RSZ, jnp.int32)
    one16 = jnp.full((16,), 1, jnp.int32)
    lists = [(l0s, l0d), (l1s, l1d)]
    bounds = [jnp.full((16,), (k + 1) * RSZ, jnp.int32) for k in range(NR)]
    bases = [jnp.full((16,), k * RSZ, jnp.int32) for k in range(NR)]

    init = tuple(jnp.zeros((16,), jnp.int32) for _ in range(NR))

    @pl.loop(0, EPP // 16, init_carry=init)
    def counts(i, carry):
        for ls, ld in lists:
            ls[pl.ds(i * 16, 16)] = zeros16
            ld[pl.ds(i * 16, 16)] = trash16
        d16 = dst_v[pl.ds(i * 16, 16)]
        s16 = src_v[pl.ds(i * 16, 16)]
        newc = []
        for k in range(NR):
            m = d16 < bounds[k]
            if k > 0:
                m = m & (d16 >= bounds[k - 1])
            pos = carry[k] + plsc.cumsum(jnp.where(m, one16, zeros16)) - one16
            plsc.store_scatter(lists[k][0], [pos], s16, mask=m)
            plsc.store_scatter(lists[k][1], [pos], d16 - bases[k], mask=m)
            newc.append(carry[k] + plsc.all_reduce_population_count(m))
        return tuple(newc)

    cns = []
    for k in range(NR):
        cnt_v[pl.ds(k * 16, 16)] = counts[k]
        cns.append(cnt_v[pl.ds(k * 16, 16)][0])

    bufs = [buf0, buf1, buf2]

    def run_pass(lsrc, ldst, cnt):
        nit = lax.max(
            lax.div(lax.shift_right_logical(cnt + (BLK - 1), 6) + 2, 3), 1)
        nb = nit * 3
        for p in range(2):
            pltpu.async_copy(h.at[lsrc.at[pl.ds(p * BLK, BLK)]],
                             bufs[p], sem)

        @pl.loop(0, nit)
        def _(it):
            for b in range(3):
                jj = it * 3 + b
                buf = bufs[b]
                nxt = bufs[(b + 2) % 3]
                pltpu.make_async_copy(
                    h.at[lsrc.at[pl.ds(jj * BLK, BLK)]], buf, sem).wait()

                @pl.when(jj + 2 < nb)
                def _():
                    pltpu.async_copy(
                        h.at[lsrc.at[pl.ds((jj + 2) * BLK, BLK)]], nxt, sem)

                pltpu.sync_copy(buf, acc.at[ldst.at[pl.ds(jj * BLK, BLK)]],
                                add=True)

    for k in range(NR):
        plsc.subcore_barrier()
        if k > 0:
            pltpu.sync_copy(zsmall, acc.at[pl.ds(s * ZROWS, ZROWS)])
            plsc.subcore_barrier()
        run_pass(lists[k][0], lists[k][1], cns[k])
        plsc.subcore_barrier()
        if k < NR - 1:
            pltpu.sync_copy(acc.at[pl.ds(s * ZROWS, ZROWS)],
                            out.at[c, pl.ds(k * RSZ + s * ZROWS, ZROWS)])
        else:
            @pl.when(s < 10)
            def _():
                rows = LASTREAL // 10
                pltpu.sync_copy(acc.at[pl.ds(s * rows, rows)],
                                out.at[c, pl.ds(k * RSZ + s * rows, rows)])


_edge_call = pl.kernel(
    _edge_body,
    out_type=jax.ShapeDtypeStruct((NC, N, D), jnp.float32),
    mesh=_mesh,
    scratch_types=[
        pltpu.VMEM((EPP,), jnp.int32),
        pltpu.VMEM((EPP,), jnp.int32),
        pltpu.VMEM((EPP,), jnp.int32),
        pltpu.VMEM((EPP,), jnp.int32),
        pltpu.VMEM((EPP,), jnp.int32),
        pltpu.VMEM((EPP,), jnp.int32),
        pltpu.VMEM((32,), jnp.int32),
        pltpu.VMEM((BLK, D), jnp.float32),
        pltpu.VMEM((BLK, D), jnp.float32),
        pltpu.VMEM((BLK, D), jnp.float32),
        pltpu.SemaphoreType.DMA,
        pltpu.VMEM_SHARED((ACCROWS, D), jnp.float32),
    ],
    compiler_params=pltpu.CompilerParams(use_tc_tiling_on_sc=False,
                                         needs_layout_passes=False),
)


_RB = 1000
_GRID = N // _RB


def _prescale_body(x_ref, dd, o_ref):
    deg = jnp.maximum(dd[...], 1.0)
    o_ref[...] = x_ref[...] * lax.rsqrt(deg)


def _attn_body(pa, pb, di, do_, w1, bb1, mh, o_ref):
    sin = lax.rsqrt(jnp.maximum(di[...], 1.0))
    agg = (pa[...] + pb[...]) * sin
    u = jnp.maximum(
        jnp.dot(agg, w1[...], preferred_element_type=jnp.float32) + bb1[...],
        0.0)
    t = jnp.dot(u * u, mh[...], preferred_element_type=jnp.float32) * 0.25
    m = jnp.max(t, axis=1, keepdims=True)
    e = jnp.exp(t - m)
    den = jnp.sum(e, axis=1, keepdims=True) * (1.0 / HD)
    att = u * (e / den)
    sout = lax.rsqrt(jnp.maximum(do_[...], 1.0))
    o_ref[...] = att * sout


def _out_body(pa, pb, di, w2, bb2, g, bt, o_ref):
    sin = lax.rsqrt(jnp.maximum(di[...], 1.0))
    agg = (pa[...] + pb[...]) * sin
    o = jnp.maximum(
        jnp.dot(agg, w2[...], preferred_element_type=jnp.float32) + bb2[...],
        0.0)
    mu = jnp.mean(o, axis=1, keepdims=True)
    xc = o - mu
    var = jnp.mean(xc * xc, axis=1, keepdims=True)
    o_ref[...] = xc * lax.rsqrt(var + 1e-5) * g[...] + bt[...]


def _row_spec(w):
    return pl.BlockSpec((_RB, w), lambda i: (i, 0))


def _full_spec(r, w):
    return pl.BlockSpec((r, w), lambda i: (0, 0))


_prescale_call = pl.pallas_call(
    _prescale_body,
    grid=(_GRID,),
    in_specs=[_row_spec(D), _row_spec(1)],
    out_specs=_row_spec(D),
    out_shape=jax.ShapeDtypeStruct((N, D), jnp.float32),
)

_attn_call = pl.pallas_call(
    _attn_body,
    grid=(_GRID,),
    in_specs=[_row_spec(D), _row_spec(D), _row_spec(1), _row_spec(1),
              _full_spec(D, D), _full_spec(1, D), _full_spec(D, D)],
    out_specs=_row_spec(D),
    out_shape=jax.ShapeDtypeStruct((N, D), jnp.float32),
)

_out_call = pl.pallas_call(
    _out_body,
    grid=(_GRID,),
    in_specs=[_row_spec(D), _row_spec(D), _row_spec(1),
              _full_spec(D, D), _full_spec(1, D),
              _full_spec(1, D), _full_spec(1, D)],
    out_specs=_row_spec(D),
    out_shape=jax.ShapeDtypeStruct((N, D), jnp.float32),
)


def _pad_edges(a, fill):
    pad = jnp.full((NW, EPP - EP), fill, jnp.int32)
    return jnp.concatenate([a.reshape(NW, EP), pad], axis=1)


def kernel(x, edge_index0, edge_index1, W1, b1, W2, b2, gamma, beta):
    src0 = edge_index0[0].astype(jnp.int32)
    dst0 = edge_index0[1].astype(jnp.int32)
    src1 = edge_index1[0].astype(jnp.int32)
    dst1 = edge_index1[1].astype(jnp.int32)

    zeros_1d = jnp.zeros((NPAD,), jnp.float32)
    zsmall = jnp.zeros((ZROWS, D), jnp.float32)

    deg_parts = _deg_call(src0, dst0, src1, dst1, zeros_1d)
    degs = _degsum_call(deg_parts.reshape(4, NW, NPAD)).reshape(4, NPAD)
    od0 = degs[0, :N].reshape(N, 1)
    id0 = degs[1, :N].reshape(N, 1)
    od1 = degs[2, :N].reshape(N, 1)
    id1 = degs[3, :N].reshape(N, 1)

    h1 = _prescale_call(x, od0)

    sb0 = _pad_edges(src0, 0)
    db0 = _pad_edges(dst0, NPAD - 1)
    sb1 = _pad_edges(src1, 0)
    db1 = _pad_edges(dst1, NPAD - 1)

    p0 = _edge_call(h1, sb0, db0, zsmall)

    hid = jnp.arange(D, dtype=jnp.int32) // HD
    mh = (hid[:, None] == hid[None, :]).astype(jnp.float32)

    h2 = _attn_call(p0[0], p0[1], id0, od1, W1, b1.reshape(1, D), mh)

    p1 = _edge_call(h2, sb1, db1, zsmall)

    return _out_call(p1[0], p1[1], id1,
                     W2, b2.reshape(1, D),
                     gamma.reshape(1, D), beta.reshape(1, D))


# speedup vs baseline: 2.5503x; 2.5503x over previous
import jax
import jax.numpy as jnp
from jax import lax
from jax.experimental import pallas as pl
from jax.experimental.pallas import tpu as pltpu
from jax.experimental.pallas import tpu_sc as plsc

N = 10000
E = 320000
D = 128
H = 8
HD = D // H

NC = 2
NS = 16
NW = NC * NS
NPAD = 10240
NROWS = NPAD // 128
EP = E // NW
BB = 128
NBB = EP // BB
TAIL = EP - NBB * BB
SROWS = N // NS
DH = D // 2
RPT = NPAD // NS
ZR = 8

_mesh = plsc.VectorSubcoreMesh(core_axis_name="c", subcore_axis_name="s")


def _deg_body(e0s, e0d, e1s, e1d, z1d, out, idx_v, acc_v):
    c = lax.axis_index("c")
    s = lax.axis_index("s")
    wid = c * NS + s
    ones = jnp.full((16,), 1.0, dtype=jnp.float32)
    for a, arr in enumerate([e0s, e0d, e1s, e1d]):
        pltpu.sync_copy(z1d, acc_v)
        pltpu.sync_copy(arr.at[pl.ds(wid * EP, EP)], idx_v)

        @pl.loop(0, EP // 16)
        def _(i):
            idx16 = idx_v[pl.ds(i * 16, 16)]
            plsc.addupdate_scatter(acc_v, [idx16], ones)

        pltpu.sync_copy(acc_v, out.at[a, c, s])


_deg_call = pl.kernel(
    _deg_body,
    out_type=jax.ShapeDtypeStruct((4, NC, NS, NPAD), jnp.float32),
    mesh=_mesh,
    scratch_types=[
        pltpu.VMEM((EP,), jnp.int32),
        pltpu.VMEM((NPAD,), jnp.float32),
    ],
    compiler_params=pltpu.CompilerParams(needs_layout_passes=False),
)


def _degsum_body(p_ref, o_ref):
    o_ref[...] = jnp.sum(p_ref[...], axis=1, keepdims=True)


_DC = 1280

_degsum_call = pl.pallas_call(
    _degsum_body,
    grid=(4, NPAD // _DC),
    in_specs=[pl.BlockSpec((1, NW, _DC), lambda a, i: (a, 0, i))],
    out_specs=pl.BlockSpec((1, 1, _DC), lambda a, i: (a, 0, i)),
    out_shape=jax.ShapeDtypeStruct((4, 1, NPAD), jnp.float32),
)


def _edge_body(h_lo, h_hi, srcb, dstb, zhalf, out,
               src_v, dst_v, buf0, buf1, tbuf, sem, shared, hstage):
    c = lax.axis_index("c")
    s = lax.axis_index("s")
    wid = c * NS + s
    pltpu.sync_copy(srcb.at[wid], src_v)
    pltpu.sync_copy(dstb.at[wid], dst_v)
    bufs = [buf0, buf1]
    for half, hh in enumerate([h_lo, h_hi]):
        pltpu.sync_copy(hh.at[pl.ds(s * SROWS, SROWS)],
                        hstage.at[pl.ds(s * SROWS, SROWS)])
        pltpu.sync_copy(zhalf.at[pl.ds(s * RPT, RPT)],
                        shared.at[pl.ds(s * RPT, RPT)])
        plsc.subcore_barrier()
        pltpu.async_copy(hstage.at[src_v.at[pl.ds(0, BB)]], buf0, sem)

        @pl.loop(0, NBB, step=2)
        def _(j):
            for b in range(2):
                jj = j + b
                buf = bufs[b]
                other = bufs[1 - b]
                pltpu.make_async_copy(
                    hstage.at[src_v.at[pl.ds(jj * BB, BB)]], buf, sem).wait()

                @pl.when(jj + 1 < NBB)
                def _():
                    pltpu.async_copy(
                        hstage.at[src_v.at[pl.ds((jj + 1) * BB, BB)]],
                        other, sem)

                pltpu.sync_copy(buf, shared.at[dst_v.at[pl.ds(jj * BB, BB)]],
                                add=True)

        pltpu.async_copy(
            hstage.at[src_v.at[pl.ds(NBB * BB, TAIL)]], tbuf, sem).wait()
        pltpu.sync_copy(tbuf, shared.at[dst_v.at[pl.ds(NBB * BB, TAIL)]],
                        add=True)

        plsc.subcore_barrier()
        pltpu.sync_copy(shared.at[pl.ds(s * RPT, RPT)],
                        out.at[c, half, pl.ds(s * RPT, RPT)])
        plsc.subcore_barrier()


_edge_call = pl.kernel(
    _edge_body,
    out_type=jax.ShapeDtypeStruct((NC, 2, NPAD, DH), jnp.float32),
    mesh=_mesh,
    scratch_types=[
        pltpu.VMEM((EP,), jnp.int32),
        pltpu.VMEM((EP,), jnp.int32),
        pltpu.VMEM((BB, DH), jnp.float32),
        pltpu.VMEM((BB, DH), jnp.float32),
        pltpu.VMEM((TAIL, DH), jnp.float32),
        pltpu.SemaphoreType.DMA,
        pltpu.VMEM_SHARED((NPAD, DH), jnp.float32),
        pltpu.VMEM_SHARED((N, DH), jnp.float32),
    ],
    compiler_params=pltpu.CompilerParams(use_tc_tiling_on_sc=False,
                                         needs_layout_passes=False),
)


_RB = 1000
_GRID = N // _RB


def _prescale_body(x_ref, dd, olo, ohi):
    deg = jnp.maximum(dd[...], 1.0)
    v = x_ref[...] * lax.rsqrt(deg)
    olo[...] = v[:, :DH]
    ohi[...] = v[:, DH:]


def _attn_body(pal, pah, pbl, pbh, di, do_, w1, bb1, mh, olo, ohi):
    sin = lax.rsqrt(jnp.maximum(di[...], 1.0))
    aggl = (pal[...] + pbl[...]) * sin
    aggh = (pah[...] + pbh[...]) * sin
    w = w1[...]
    u = jnp.maximum(
        jnp.dot(aggl, w[:DH], preferred_element_type=jnp.float32)
        + jnp.dot(aggh, w[DH:], preferred_element_type=jnp.float32)
        + bb1[...], 0.0)
    t = jnp.dot(u * u, mh[...], preferred_element_type=jnp.float32) * 0.25
    m = jnp.max(t, axis=1, keepdims=True)
    e = jnp.exp(t - m)
    den = jnp.sum(e, axis=1, keepdims=True) * (1.0 / HD)
    att = u * (e / den)
    sout = lax.rsqrt(jnp.maximum(do_[...], 1.0))
    h2 = att * sout
    olo[...] = h2[:, :DH]
    ohi[...] = h2[:, DH:]


def _out_body(pal, pah, pbl, pbh, di, w2, bb2, g, bt, o_ref):
    sin = lax.rsqrt(jnp.maximum(di[...], 1.0))
    aggl = (pal[...] + pbl[...]) * sin
    aggh = (pah[...] + pbh[...]) * sin
    w = w2[...]
    o = jnp.maximum(
        jnp.dot(aggl, w[:DH], preferred_element_type=jnp.float32)
        + jnp.dot(aggh, w[DH:], preferred_element_type=jnp.float32)
        + bb2[...], 0.0)
    mu = jnp.mean(o, axis=1, keepdims=True)
    xc = o - mu
    var = jnp.mean(xc * xc, axis=1, keepdims=True)
    o_ref[...] = xc * lax.rsqrt(var + 1e-5) * g[...] + bt[...]


def _row_spec(w):
    return pl.BlockSpec((_RB, w), lambda i: (i, 0))


def _full_spec(r, w):
    return pl.BlockSpec((r, w), lambda i: (0, 0))


_prescale_call = pl.pallas_call(
    _prescale_body,
    grid=(_GRID,),
    in_specs=[_row_spec(D), _row_spec(1)],
    out_specs=(_row_spec(DH), _row_spec(DH)),
    out_shape=(jax.ShapeDtypeStruct((N, DH), jnp.float32),
               jax.ShapeDtypeStruct((N, DH), jnp.float32)),
)

_attn_call = pl.pallas_call(
    _attn_body,
    grid=(_GRID,),
    in_specs=[_row_spec(DH), _row_spec(DH), _row_spec(DH), _row_spec(DH),
              _row_spec(1), _row_spec(1),
              _full_spec(D, D), _full_spec(1, D), _full_spec(D, D)],
    out_specs=(_row_spec(DH), _row_spec(DH)),
    out_shape=(jax.ShapeDtypeStruct((N, DH), jnp.float32),
               jax.ShapeDtypeStruct((N, DH), jnp.float32)),
)

_out_call = pl.pallas_call(
    _out_body,
    grid=(_GRID,),
    in_specs=[_row_spec(DH), _row_spec(DH), _row_spec(DH), _row_spec(DH),
              _row_spec(1),
              _full_spec(D, D), _full_spec(1, D),
              _full_spec(1, D), _full_spec(1, D)],
    out_specs=_row_spec(D),
    out_shape=jax.ShapeDtypeStruct((N, D), jnp.float32),
)


def _shape_edges(a):
    return a.reshape(NW, EP)


def kernel(x, edge_index0, edge_index1, W1, b1, W2, b2, gamma, beta):
    src0 = edge_index0[0].astype(jnp.int32)
    dst0 = edge_index0[1].astype(jnp.int32)
    src1 = edge_index1[0].astype(jnp.int32)
    dst1 = edge_index1[1].astype(jnp.int32)

    zeros_1d = jnp.zeros((NPAD,), jnp.float32)
    zhalf = jnp.zeros((NPAD, DH), jnp.float32)

    deg_parts = _deg_call(src0, dst0, src1, dst1, zeros_1d)
    degs = _degsum_call(deg_parts.reshape(4, NW, NPAD)).reshape(4, NPAD)
    od0 = degs[0, :N].reshape(N, 1)
    id0 = degs[1, :N].reshape(N, 1)
    od1 = degs[2, :N].reshape(N, 1)
    id1 = degs[3, :N].reshape(N, 1)

    h1lo, h1hi = _prescale_call(x, od0)

    sb0 = _shape_edges(src0)
    db0 = _shape_edges(dst0)
    sb1 = _shape_edges(src1)
    db1 = _shape_edges(dst1)

    p0 = _edge_call(h1lo, h1hi, sb0, db0, zhalf)

    hid = jnp.arange(D, dtype=jnp.int32) // HD
    mh = (hid[:, None] == hid[None, :]).astype(jnp.float32)

    h2lo, h2hi = _attn_call(p0[0, 0, :N], p0[0, 1, :N],
                            p0[1, 0, :N], p0[1, 1, :N],
                            id0, od1, W1, b1.reshape(1, D), mh)

    p1 = _edge_call(h2lo, h2hi, sb1, db1, zhalf)

    return _out_call(p1[0, 0, :N], p1[0, 1, :N],
                     p1[1, 0, :N], p1[1, 1, :N], id1,
                     W2, b2.reshape(1, D),
                     gamma.reshape(1, D), beta.reshape(1, D))
